# Initial kernel scaffold; baseline (speedup 1.0000x reference)
import jax
import jax.numpy as jnp
from jax.experimental import pallas as pl


def kernel(vertices, edges, messages):
    raise NotImplementedError("write your pallas kernel here")


# speedup vs baseline: 7.5330x; 7.5330x over previous
import functools

import jax
import jax.numpy as jnp
from jax import lax
from jax.experimental import pallas as pl
from jax.experimental.pallas import tpu as pltpu
from jax.experimental.pallas import tpu_sc as plsc

N_NODES = 100000
N_EDGES = 3200000
NPAD = 102400
EPS = 1e-6

NB = N_EDGES // 512
BPW = 196
RPT = NPAD // 16
FBLK = NPAD // 128 // 32

_mesh = plsc.VectorSubcoreMesh(core_axis_name="c", subcore_axis_name="s")

_f32 = jnp.float32
_i32 = jnp.int32


def _rsqrt(x):
    i = lax.bitcast_convert_type(x, _i32)
    i = jnp.int32(0x5F3759DF) - (i >> 1)
    y = lax.bitcast_convert_type(i, _f32)
    y = y * (1.5 - 0.5 * x * y * y)
    y = y * (1.5 - 0.5 * x * y * y)
    return y


def _log(x):
    bits = lax.bitcast_convert_type(x, _i32)
    e = (bits >> 23) - 127
    m = lax.bitcast_convert_type((bits & 0x7FFFFF) | 0x3F800000, _f32)
    big = m > 1.4142135
    m = jnp.where(big, m * 0.5, m)
    ef = (e + big.astype(_i32)).astype(_f32)
    t = (m - 1.0) / (m + 1.0)
    t2 = t * t
    p = 2.0 * t * (1.0 + t2 * (1.0 / 3.0 + t2 * (0.2 + t2 * (1.0 / 7.0))))
    return ef * 0.6931471805599453 + p


def _dg(v, idx):
    return lax.gather(
        v, idx[:, None],
        lax.GatherDimensionNumbers(offset_dims=(), collapsed_slice_dims=(0,),
                                   start_index_map=(0,)),
        (1,), mode=lax.GatherScatterMode.PROMISE_IN_BOUNDS)


_PART = jax.ShapeDtypeStruct((NPAD,), _f32)


@functools.partial(
    pl.kernel,
    mesh=_mesh,
    compiler_params=pltpu.CompilerParams(needs_layout_passes=False),
    out_type=[_PART] * 8,
    scratch_types=(
        [pltpu.VMEM_SHARED((NPAD,), _f32)] * 7
        + [pltpu.VMEM((128,), _i32)] * 8
        + [pltpu.VMEM((2048,), _f32)]
        + [pltpu.VMEM((128,), _f32)] * 10
        + [pltpu.SemaphoreType.DMA]
    ),
)
def _agg(vt_h, v1_h, v2_h, src_h, dst_h, msg_h, z_h,
         s0t, s01, s02, s0c, s1t, s11, s12, s1c,
         vt, v1, v2, at, a1, a2, ac,
         si0, si1, si2, si3, di0, di1, di2, di3, msgv,
         xt_v, x1_v, x2_v, yt_v, y1_v, y2_v, ot_v, o1_v, o2_v, ones_v,
         sem):
    c = lax.axis_index("c")
    s = lax.axis_index("s")
    wid = s * 2 + c
    lo = s * RPT
    sl = pl.ds(lo, RPT)
    pltpu.sync_copy(vt_h.at[sl], vt.at[sl])
    pltpu.sync_copy(v1_h.at[sl], v1.at[sl])
    pltpu.sync_copy(v2_h.at[sl], v2.at[sl])
    pltpu.sync_copy(z_h, at.at[sl])
    pltpu.sync_copy(z_h, a1.at[sl])
    pltpu.sync_copy(z_h, a2.at[sl])
    pltpu.sync_copy(z_h, ac.at[sl])
    plsc.subcore_barrier()

    iota = lax.iota(_i32, 16)
    i1 = iota ^ 1
    i2 = iota ^ 2
    pidx = (iota & 3) * 4
    m0 = iota < 4
    m1 = iota < 8
    m2 = iota < 12
    one16 = jnp.ones((16,), _f32)
    for g in range(8):
        ones_v[pl.ds(g * 16, 16)] = one16

    def block(i, carry):
        b = wid * BPW + i

        @pl.when(b < NB)
        def _():
            sbufs = [si0, si1, si2, si3]
            dbufs = [di0, di1, di2, di3]
            for j in range(4):
                pltpu.sync_copy(src_h.at[pl.ds(b * 512 + j * 128, 128)], sbufs[j])
                pltpu.sync_copy(dst_h.at[pl.ds(b * 512 + j * 128, 128)], dbufs[j])
            pltpu.sync_copy(msg_h.at[pl.ds(b * 2048, 2048)], msgv)
            for j in range(4):
                sj = sbufs[j]
                dj = dbufs[j]
                cps = [pltpu.async_copy(vt.at[sj], xt_v, sem),
                       pltpu.async_copy(v1.at[sj], x1_v, sem),
                       pltpu.async_copy(v2.at[sj], x2_v, sem),
                       pltpu.async_copy(vt.at[dj], yt_v, sem),
                       pltpu.async_copy(v1.at[dj], y1_v, sem),
                       pltpu.async_copy(v2.at[dj], y2_v, sem)]
                for cp in cps:
                    cp.wait()
                for g in range(8):
                    gs = pl.ds(g * 16, 16)
                    us = []
                    for k in range(4):
                        v = msgv[pl.ds(j * 512 + g * 64 + k * 16, 16)]
                        u = v + _dg(v, i1)
                        us.append(u + _dg(u, i2))
                    w = jnp.where(
                        m0, _dg(us[0], pidx),
                        jnp.where(m1, _dg(us[1], pidx),
                                  jnp.where(m2, _dg(us[2], pidx),
                                            _dg(us[3], pidx)))) * 0.25
                    xt = xt_v[gs]
                    x1 = x1_v[gs]
                    x2 = x2_v[gs]
                    yt = yt_v[gs]
                    y1 = y1_v[gs]
                    y2 = y2_v[gs]
                    t = x1 * y1 + x2 * y2 - xt * yt
                    ot = yt + xt * t
                    o1 = y1 + x1 * t
                    o2 = y2 + x2 * t
                    q = o1 * o1 + o2 * o2 - ot * ot + EPS
                    rinv = _rsqrt(q)
                    arg = jnp.maximum(-t, 1.000001)
                    s2 = (arg - 1.0) * (arg + 1.0)
                    dist = _log(arg + s2 * _rsqrt(s2))
                    sc = w * dist * rinv
                    ot_v[gs] = ot * sc
                    o1_v[gs] = o1 * sc
                    o2_v[gs] = o2 * sc
                pltpu.sync_copy(ot_v, at.at[sj], add=True)
                pltpu.sync_copy(o1_v, a1.at[sj], add=True)
                pltpu.sync_copy(o2_v, a2.at[sj], add=True)
                pltpu.sync_copy(ones_v, ac.at[sj], add=True)
        return carry

    lax.fori_loop(0, BPW, block, 0)
    plsc.subcore_barrier()

    @pl.when(c == 0)
    def _():
        pltpu.sync_copy(at.at[sl], s0t.at[sl])
        pltpu.sync_copy(a1.at[sl], s01.at[sl])
        pltpu.sync_copy(a2.at[sl], s02.at[sl])
        pltpu.sync_copy(ac.at[sl], s0c.at[sl])

    @pl.when(c == 1)
    def _():
        pltpu.sync_copy(at.at[sl], s1t.at[sl])
        pltpu.sync_copy(a1.at[sl], s11.at[sl])
        pltpu.sync_copy(a2.at[sl], s12.at[sl])
        pltpu.sync_copy(ac.at[sl], s1c.at[sl])


@functools.partial(
    pl.kernel,
    mesh=_mesh,
    compiler_params=pltpu.CompilerParams(needs_layout_passes=False),
    out_type=[_PART] * 3,
    scratch_types=(
        [pltpu.VMEM((128,), _f32)] * 14
        + [pltpu.SemaphoreType.DMA]
    ),
)
def _exp(s0t, s01, s02, s0c, s1t, s11, s12, s1c, vt_h, v1_h, v2_h,
         o0_h, o1_h, o2_h,
         b0t, b01, b02, b0c, b1t, b11, b12, b1c, bvt, bv1, bv2,
         ob0, ob1, ob2, sem):
    c = lax.axis_index("c")
    s = lax.axis_index("s")
    wid = s * 2 + c

    def block(i, carry):
        base = (wid * FBLK + i) * 128
        bs = pl.ds(base, 128)
        cps = [pltpu.async_copy(s0t.at[bs], b0t, sem),
               pltpu.async_copy(s01.at[bs], b01, sem),
               pltpu.async_copy(s02.at[bs], b02, sem),
               pltpu.async_copy(s0c.at[bs], b0c, sem),
               pltpu.async_copy(s1t.at[bs], b1t, sem),
               pltpu.async_copy(s11.at[bs], b11, sem),
               pltpu.async_copy(s12.at[bs], b12, sem),
               pltpu.async_copy(s1c.at[bs], b1c, sem),
               pltpu.async_copy(vt_h.at[bs], bvt, sem),
               pltpu.async_copy(v1_h.at[bs], bv1, sem),
               pltpu.async_copy(v2_h.at[bs], bv2, sem)]
        for cp in cps:
            cp.wait()
        for g in range(8):
            gs = pl.ds(g * 16, 16)
            t0 = b0t[gs] + b1t[gs]
            t1 = b01[gs] + b11[gs]
            t2 = b02[gs] + b12[gs]
            cnt = b0c[gs] + b1c[gs]
            inv = 1.0 / jnp.maximum(cnt, 1.0)
            pos = cnt > 0.0
            t0 = jnp.where(pos, t0 * inv, 0.0)
            t1 = jnp.where(pos, t1 * inv, 0.0)
            t2 = jnp.where(pos, t2 * inv, 0.0)
            q = t1 * t1 + t2 * t2 - t0 * t0 + EPS
            r2 = _rsqrt(q)
            T = q * r2
            ee = jnp.exp(T)
            ei = 1.0 / ee
            ch = (ee + ei) * 0.5
            sh = (ee - ei) * 0.5
            ob0[gs] = ch * bvt[gs] + sh * (t0 * r2)
            ob1[gs] = ch * bv1[gs] + sh * (t1 * r2)
            ob2[gs] = ch * bv2[gs] + sh * (t2 * r2)
        pltpu.sync_copy(ob0, o0_h.at[bs])
        pltpu.sync_copy(ob1, o1_h.at[bs])
        pltpu.sync_copy(ob2, o2_h.at[bs])
        return carry

    lax.fori_loop(0, FBLK, block, 0)


def kernel(vertices, edges, messages):
    pad = (0, NPAD - N_NODES)
    vt = jnp.pad(vertices[:, 0], pad)
    v1 = jnp.pad(vertices[:, 1], pad)
    v2 = jnp.pad(vertices[:, 2], pad)
    src1 = edges[:, 0]
    dst1 = edges[:, 1]
    msgf = messages.reshape(-1)
    z = jnp.zeros((RPT,), _f32)
    parts = _agg(vt, v1, v2, src1, dst1, msgf, z)
    o0, o1, o2 = _exp(*parts, vt, v1, v2)
    return jnp.stack([o0[:N_NODES], o1[:N_NODES], o2[:N_NODES]], axis=1)


# speedup vs baseline: 8.5327x; 1.1327x over previous
import functools

import jax
import jax.numpy as jnp
from jax import lax
from jax.experimental import pallas as pl
from jax.experimental.pallas import tpu as pltpu
from jax.experimental.pallas import tpu_sc as plsc

N_NODES = 100000
N_EDGES = 3200000
NPAD = 102400
EPS = 1e-6

NB = N_EDGES // 512
BPW = 196
RPT = NPAD // 16
FBLK = NPAD // 128 // 32

_mesh = plsc.VectorSubcoreMesh(core_axis_name="c", subcore_axis_name="s")

_f32 = jnp.float32
_i32 = jnp.int32


def _rsqrt(x):
    i = lax.bitcast_convert_type(x, _i32)
    i = jnp.int32(0x5F3759DF) - (i >> 1)
    y = lax.bitcast_convert_type(i, _f32)
    y = y * (1.5 - 0.5 * x * y * y)
    y = y * (1.5 - 0.5 * x * y * y)
    return y


def _log(x):
    bits = lax.bitcast_convert_type(x, _i32)
    e = (bits >> 23) - 127
    m = lax.bitcast_convert_type((bits & 0x7FFFFF) | 0x3F800000, _f32)
    big = m > 1.4142135
    m = jnp.where(big, m * 0.5, m)
    ef = (e + big.astype(_i32)).astype(_f32)
    t = (m - 1.0) / (m + 1.0)
    t2 = t * t
    p = 2.0 * t * (1.0 + t2 * (1.0 / 3.0 + t2 * (0.2 + t2 * (1.0 / 7.0))))
    return ef * 0.6931471805599453 + p


def _dg(v, idx):
    return lax.gather(
        v, idx[:, None],
        lax.GatherDimensionNumbers(offset_dims=(), collapsed_slice_dims=(0,),
                                   start_index_map=(0,)),
        (1,), mode=lax.GatherScatterMode.PROMISE_IN_BOUNDS)


_PART = jax.ShapeDtypeStruct((NPAD,), _f32)


@functools.partial(
    pl.kernel,
    mesh=_mesh,
    compiler_params=pltpu.CompilerParams(needs_layout_passes=False),
    out_type=[_PART] * 8,
    scratch_types=(
        [pltpu.VMEM_SHARED((NPAD,), _f32)] * 7
        + [pltpu.VMEM((512,), _i32)] * 2
        + [pltpu.VMEM((2048,), _f32)]
        + [pltpu.VMEM((512,), _f32)] * 10
        + [pltpu.SemaphoreType.DMA] * 2
    ),
)
def _agg(vt_h, v1_h, v2_h, src_h, dst_h, msg_h, z_h,
         s0t, s01, s02, s0c, s1t, s11, s12, s1c,
         vt, v1, v2, at, a1, a2, ac,
         si, di, msgv,
         xt_v, x1_v, x2_v, yt_v, y1_v, y2_v, ot_v, o1_v, o2_v, ones_v,
         sem, sem2):
    c = lax.axis_index("c")
    s = lax.axis_index("s")
    wid = s * 2 + c
    lo = s * RPT
    sl = pl.ds(lo, RPT)
    pltpu.sync_copy(vt_h.at[sl], vt.at[sl])
    pltpu.sync_copy(v1_h.at[sl], v1.at[sl])
    pltpu.sync_copy(v2_h.at[sl], v2.at[sl])
    pltpu.sync_copy(z_h, at.at[sl])
    pltpu.sync_copy(z_h, a1.at[sl])
    pltpu.sync_copy(z_h, a2.at[sl])
    pltpu.sync_copy(z_h, ac.at[sl])
    plsc.subcore_barrier()

    iota = lax.iota(_i32, 16)
    i1 = iota ^ 1
    i2 = iota ^ 2
    pidx = (iota & 3) * 4
    m0 = iota < 4
    m1 = iota < 8
    m2 = iota < 12
    one16 = jnp.ones((16,), _f32)
    for g in range(32):
        ones_v[pl.ds(g * 16, 16)] = one16

    def block(i, carry):
        b = wid * BPW + i

        @pl.when(b < NB)
        def _():
            pltpu.sync_copy(src_h.at[pl.ds(b * 512, 512)], si)
            pltpu.sync_copy(dst_h.at[pl.ds(b * 512, 512)], di)
            pltpu.sync_copy(msg_h.at[pl.ds(b * 2048, 2048)], msgv)
            cps = [pltpu.async_copy(vt.at[si], xt_v, sem),
                   pltpu.async_copy(v1.at[si], x1_v, sem),
                   pltpu.async_copy(v2.at[si], x2_v, sem),
                   pltpu.async_copy(vt.at[di], yt_v, sem),
                   pltpu.async_copy(v1.at[di], y1_v, sem),
                   pltpu.async_copy(v2.at[di], y2_v, sem)]
            for cp in cps:
                cp.wait()
            for g in range(32):
                gs = pl.ds(g * 16, 16)
                us = []
                for k in range(4):
                    v = msgv[pl.ds(g * 64 + k * 16, 16)]
                    u = v + _dg(v, i1)
                    us.append(u + _dg(u, i2))
                w = jnp.where(
                    m0, _dg(us[0], pidx),
                    jnp.where(m1, _dg(us[1], pidx),
                              jnp.where(m2, _dg(us[2], pidx),
                                        _dg(us[3], pidx)))) * 0.25
                xt = xt_v[gs]
                x1 = x1_v[gs]
                x2 = x2_v[gs]
                yt = yt_v[gs]
                y1 = y1_v[gs]
                y2 = y2_v[gs]
                t = x1 * y1 + x2 * y2 - xt * yt
                ot = yt + xt * t
                o1 = y1 + x1 * t
                o2 = y2 + x2 * t
                q = o1 * o1 + o2 * o2 - ot * ot + EPS
                rinv = _rsqrt(q)
                arg = jnp.maximum(-t, 1.000001)
                s2 = (arg - 1.0) * (arg + 1.0)
                dist = _log(arg + s2 * _rsqrt(s2))
                sc = w * dist * rinv
                ot_v[gs] = ot * sc
                o1_v[gs] = o1 * sc
                o2_v[gs] = o2 * sc
            wps = [pltpu.async_copy(ot_v, at.at[si], sem2, add=True),
                   pltpu.async_copy(o1_v, a1.at[si], sem2, add=True),
                   pltpu.async_copy(o2_v, a2.at[si], sem2, add=True),
                   pltpu.async_copy(ones_v, ac.at[si], sem2, add=True)]
            for wp in wps:
                wp.wait()
        return carry

    lax.fori_loop(0, BPW, block, 0)
    plsc.subcore_barrier()

    @pl.when(c == 0)
    def _():
        pltpu.sync_copy(at.at[sl], s0t.at[sl])
        pltpu.sync_copy(a1.at[sl], s01.at[sl])
        pltpu.sync_copy(a2.at[sl], s02.at[sl])
        pltpu.sync_copy(ac.at[sl], s0c.at[sl])

    @pl.when(c == 1)
    def _():
        pltpu.sync_copy(at.at[sl], s1t.at[sl])
        pltpu.sync_copy(a1.at[sl], s11.at[sl])
        pltpu.sync_copy(a2.at[sl], s12.at[sl])
        pltpu.sync_copy(ac.at[sl], s1c.at[sl])


@functools.partial(
    pl.kernel,
    mesh=_mesh,
    compiler_params=pltpu.CompilerParams(needs_layout_passes=False),
    out_type=[_PART] * 3,
    scratch_types=(
        [pltpu.VMEM((128,), _f32)] * 14
        + [pltpu.SemaphoreType.DMA]
    ),
)
def _exp(s0t, s01, s02, s0c, s1t, s11, s12, s1c, vt_h, v1_h, v2_h,
         o0_h, o1_h, o2_h,
         b0t, b01, b02, b0c, b1t, b11, b12, b1c, bvt, bv1, bv2,
         ob0, ob1, ob2, sem):
    c = lax.axis_index("c")
    s = lax.axis_index("s")
    wid = s * 2 + c

    def block(i, carry):
        base = (wid * FBLK + i) * 128
        bs = pl.ds(base, 128)
        cps = [pltpu.async_copy(s0t.at[bs], b0t, sem),
               pltpu.async_copy(s01.at[bs], b01, sem),
               pltpu.async_copy(s02.at[bs], b02, sem),
               pltpu.async_copy(s0c.at[bs], b0c, sem),
               pltpu.async_copy(s1t.at[bs], b1t, sem),
               pltpu.async_copy(s11.at[bs], b11, sem),
               pltpu.async_copy(s12.at[bs], b12, sem),
               pltpu.async_copy(s1c.at[bs], b1c, sem),
               pltpu.async_copy(vt_h.at[bs], bvt, sem),
               pltpu.async_copy(v1_h.at[bs], bv1, sem),
               pltpu.async_copy(v2_h.at[bs], bv2, sem)]
        for cp in cps:
            cp.wait()
        for g in range(8):
            gs = pl.ds(g * 16, 16)
            t0 = b0t[gs] + b1t[gs]
            t1 = b01[gs] + b11[gs]
            t2 = b02[gs] + b12[gs]
            cnt = b0c[gs] + b1c[gs]
            inv = 1.0 / jnp.maximum(cnt, 1.0)
            pos = cnt > 0.0
            t0 = jnp.where(pos, t0 * inv, 0.0)
            t1 = jnp.where(pos, t1 * inv, 0.0)
            t2 = jnp.where(pos, t2 * inv, 0.0)
            q = t1 * t1 + t2 * t2 - t0 * t0 + EPS
            r2 = _rsqrt(q)
            T = q * r2
            ee = jnp.exp(T)
            ei = 1.0 / ee
            ch = (ee + ei) * 0.5
            sh = (ee - ei) * 0.5
            ob0[gs] = ch * bvt[gs] + sh * (t0 * r2)
            ob1[gs] = ch * bv1[gs] + sh * (t1 * r2)
            ob2[gs] = ch * bv2[gs] + sh * (t2 * r2)
        pltpu.sync_copy(ob0, o0_h.at[bs])
        pltpu.sync_copy(ob1, o1_h.at[bs])
        pltpu.sync_copy(ob2, o2_h.at[bs])
        return carry

    lax.fori_loop(0, FBLK, block, 0)


def kernel(vertices, edges, messages):
    pad = (0, NPAD - N_NODES)
    vt = jnp.pad(vertices[:, 0], pad)
    v1 = jnp.pad(vertices[:, 1], pad)
    v2 = jnp.pad(vertices[:, 2], pad)
    src1 = edges[:, 0]
    dst1 = edges[:, 1]
    msgf = messages.reshape(-1)
    z = jnp.zeros((RPT,), _f32)
    parts = _agg(vt, v1, v2, src1, dst1, msgf, z)
    o0, o1, o2 = _exp(*parts, vt, v1, v2)
    return jnp.stack([o0[:N_NODES], o1[:N_NODES], o2[:N_NODES]], axis=1)
